# baseline (speedup 1.0000x reference)
import functools

import jax
import jax.numpy as jnp
from jax import lax
from jax.experimental import pallas as pl
from jax.experimental.pallas import tpu as pltpu
from jax.experimental.pallas import tpu_sc as plsc

L = 16
NC = 2
NS = 16
NW = NC * NS
T = 8
D = 16
B = 4
S = 8192
N_TOK = B * S
BLK = 128
NBLK = N_TOK // BLK
BPW = NBLK // NW
NG = BLK // L
ST = S // BLK
NPAR = 5 + 2 * T
WSPLIT = 4


def _full(v, dtype=jnp.int32):
    return jnp.full((L,), v, dtype)


def _bf16_round(v):
    t = v * 65537.0
    return t - (t - v)


def _sc_router_body(x_hbm, pos_hbm, par_hbm,
                    sel_hbm, tgt_hbm, ps_hbm, cs_hbm, cb_hbm,
                    xv0, xv1, pv, parv, wsp, psv, csv, cbv, selv, tgtv,
                    sem, sem_out):
    wid = lax.axis_index("s") * NC + lax.axis_index("c")
    b = wid // (ST // BPW)
    st0 = (wid % (ST // BPW)) * BPW

    cps = [
        pltpu.async_copy(par_hbm, parv, sem),
        pltpu.async_copy(x_hbm.at[pl.ds((b * 2 * ST + st0) * (8 * BLK),
                                        BPW * 8 * BLK)], xv0, sem),
        pltpu.async_copy(x_hbm.at[pl.ds(((b * 2 + 1) * ST + st0) * (8 * BLK),
                                        BPW * 8 * BLK)], xv1, sem),
    ]
    for i in range(BPW):
        cps.append(pltpu.async_copy(
            pos_hbm.at[pl.ds(((st0 + i) * B + b) * BLK, BLK)],
            pv.at[pl.ds(i * BLK, BLK)], sem))
    for cp in cps:
        cp.wait()

    def prow(i):
        return parv[pl.ds(i * L, L)]

    thr = prow(0)
    pw = prow(1)
    cw = prow(2)
    pearly = prow(3)
    plate = prow(4)

    wsc = []
    for t in range(T):
        wt = prow(5 + t)
        wsc.append([wt[c] for c in range(D)])
    for t in range(WSPLIT):
        for c in range(D):
            wsp[pl.ds((t * D + c) * L, L)] = jnp.full((L,), wsc[t][c])

    esp, lsp = [], []
    for t in range(T):
        wt = prow(5 + T + t)
        esp.append(jnp.full((L,), jnp.sum(wt * pearly)))
        lsp.append(jnp.full((L,), jnp.sum(wt * plate)))

    def block(i, carry):
        @plsc.parallel_loop(0, NG)
        def group(q):
            xb = i * (8 * BLK) + q * L
            pvec = pv[pl.ds(i * BLK + q * L, L)]
            mask = pvec.astype(jnp.float32) < thr

            xT0 = xv0[pl.ds(xb, L)]
            xT1 = xv0[pl.ds(xb + BLK, L)]
            xR = []
            for c in range(D):
                src = xv0 if c < 8 else xv1
                xR.append(_bf16_round(src[pl.ds(xb + (c % 8) * BLK, L)]))

            ob = i * (T * BLK) + q * L
            best = None
            bidx = None
            for t in range(T):
                if t < WSPLIT:
                    acc = xR[0] * wsp[pl.ds((t * D) * L, L)]
                    for c in range(1, D):
                        acc = acc + xR[c] * wsp[pl.ds((t * D + c) * L, L)]
                else:
                    acc = xR[0] * wsc[t][0]
                    for c in range(1, D):
                        acc = acc + xR[c] * wsc[t][c]
                post = jnp.where(mask, esp[t], lsp[t])
                comb = pw * post + cw * acc
                psv[pl.ds(ob + t * BLK, L)] = post
                csv[pl.ds(ob + t * BLK, L)] = acc
                cbv[pl.ds(ob + t * BLK, L)] = comb
                if t == 0:
                    best, bidx = comb, _full(0)
                else:
                    gt = comb > best
                    best = jnp.where(gt, comb, best)
                    bidx = jnp.where(gt, _full(t), bidx)

            pos_class = jnp.where(mask, _full(0), _full(1))
            f0 = (xT0 > 0).astype(jnp.int32)
            f1 = (xT1 > 0).astype(jnp.int32)
            selv[pl.ds(i * BLK + q * L, L)] = bidx
            tgtv[pl.ds(i * BLK + q * L, L)] = pos_class * 4 + f0 * 2 + f1

        off = ((st0 + i) * B + b) * BLK
        sc_off = (wid * BPW + i) * (T * BLK)
        for src, dst in ((psv, ps_hbm), (csv, cs_hbm), (cbv, cb_hbm)):
            pltpu.async_copy(src.at[pl.ds(i * (T * BLK), T * BLK)],
                             dst.at[pl.ds(sc_off, T * BLK)], sem_out)
        pltpu.async_copy(selv.at[pl.ds(i * BLK, BLK)],
                         sel_hbm.at[pl.ds(off, BLK)], sem_out)
        pltpu.async_copy(tgtv.at[pl.ds(i * BLK, BLK)],
                         tgt_hbm.at[pl.ds(off, BLK)], sem_out)
        return carry

    lax.fori_loop(0, BPW, block, 0)

    for i in range(BPW):
        off = ((st0 + i) * B + b) * BLK
        sc_off = (wid * BPW + i) * (T * BLK)
        for src, dst in ((psv, ps_hbm), (csv, cs_hbm), (cbv, cb_hbm)):
            pltpu.make_async_copy(src.at[pl.ds(i * (T * BLK), T * BLK)],
                                  dst.at[pl.ds(sc_off, T * BLK)],
                                  sem_out).wait()
        pltpu.make_async_copy(selv.at[pl.ds(i * BLK, BLK)],
                              sel_hbm.at[pl.ds(off, BLK)], sem_out).wait()
        pltpu.make_async_copy(tgtv.at[pl.ds(i * BLK, BLK)],
                              tgt_hbm.at[pl.ds(off, BLK)], sem_out).wait()


_OUT_TYPE = (
    jax.ShapeDtypeStruct((N_TOK,), jnp.int32),
    jax.ShapeDtypeStruct((N_TOK,), jnp.int32),
    jax.ShapeDtypeStruct((N_TOK * T,), jnp.float32),
    jax.ShapeDtypeStruct((N_TOK * T,), jnp.float32),
    jax.ShapeDtypeStruct((N_TOK * T,), jnp.float32),
)

_SCRATCH = (
    pltpu.VMEM((BPW * 8 * BLK,), jnp.float32),
    pltpu.VMEM((BPW * 8 * BLK,), jnp.float32),
    pltpu.VMEM((BPW * BLK,), jnp.int32),
    pltpu.VMEM((NPAR * L,), jnp.float32),
    pltpu.VMEM((WSPLIT * D * L,), jnp.float32),
    pltpu.VMEM((BPW * T * BLK,), jnp.float32),
    pltpu.VMEM((BPW * T * BLK,), jnp.float32),
    pltpu.VMEM((BPW * T * BLK,), jnp.float32),
    pltpu.VMEM((BPW * BLK,), jnp.int32),
    pltpu.VMEM((BPW * BLK,), jnp.int32),
    pltpu.SemaphoreType.DMA,
    pltpu.SemaphoreType.DMA,
)


@functools.lru_cache(maxsize=None)
def _sc_router():
    return pl.kernel(
        _sc_router_body,
        out_type=_OUT_TYPE,
        mesh=plsc.VectorSubcoreMesh(core_axis_name="c", subcore_axis_name="s",
                                    num_cores=NC, num_subcores=NS),
        scratch_types=_SCRATCH,
        compiler_params=pltpu.CompilerParams(
            needs_layout_passes=False,
            disable_bounds_checks=True,
            disable_semaphore_checks=True,
            skip_device_barrier=True,
        ),
    )


def _b16(v):
    y = lax.bitcast_convert_type(v, jnp.int32)
    odd = lax.shift_right_logical(y, 16) & 1
    r = (y + 32767 + odd) & (-65536)
    return lax.bitcast_convert_type(r, jnp.float32)


def kernel(x, positions, seq_len, position_sigs, content_sigs,
           position_logit, content_logit, pos_early, pos_late):
    xf = (x.astype(jnp.float32)
          .transpose(0, 2, 1)
          .reshape(B, 2, 8, ST, BLK)
          .transpose(0, 1, 3, 2, 4)
          .reshape(N_TOK * D))
    pf = (positions.astype(jnp.int32)
          .reshape(B, ST, BLK)
          .transpose(1, 0, 2)
          .reshape(N_TOK))
    half = jnp.asarray(seq_len, jnp.float32) / 2.0
    sp = jax.nn.sigmoid(jnp.asarray(position_logit, jnp.float32))
    sc = jax.nn.sigmoid(jnp.asarray(content_logit, jnp.float32))
    raw = jnp.concatenate([
        jnp.full((L,), half, jnp.float32),
        jnp.full((L,), sp / (sp + sc), jnp.float32),
        jnp.full((L,), sc / (sp + sc), jnp.float32),
        pos_early.astype(jnp.float32),
        pos_late.astype(jnp.float32),
        content_sigs.astype(jnp.float32).reshape(-1),
        position_sigs.astype(jnp.float32).reshape(-1),
    ])
    row = lax.iota(jnp.int32, NPAR * L) // L
    v = jnp.where(row >= 5, jnp.tanh(raw), raw)
    params = jnp.where(row >= 3, _b16(v), v)

    sel, tgt, ps, cs, cb = _sc_router()(xf, pf, params)

    def untile_tok(v):
        return v.reshape(ST, B, BLK).transpose(1, 0, 2).reshape(B, S)

    def untile_scores(v):
        return (v.reshape(B, ST, T, BLK)
                .transpose(0, 1, 3, 2)
                .reshape(B, S, T))

    return (untile_tok(sel), untile_tok(tgt),
            untile_scores(ps), untile_scores(cs), untile_scores(cb))


# speedup vs baseline: 1.1152x; 1.1152x over previous
import functools

import jax
import jax.numpy as jnp
from jax import lax
from jax.experimental import pallas as pl
from jax.experimental.pallas import tpu as pltpu
from jax.experimental.pallas import tpu_sc as plsc

L = 16
NC = 2
NS = 16
NW = NC * NS
T = 8
D = 16
B = 4
S = 8192
N_TOK = B * S
BLK = 128
NBLK = N_TOK // BLK
BPW = NBLK // NW
NG = BLK // L
ST = S // BLK
NPAR = 5 + 2 * T
WSPLIT = 4


def _full(v, dtype=jnp.int32):
    return jnp.full((L,), v, dtype)


def _bf16_round(v):
    t = v * 65537.0
    return t - (t - v)


def _sc_router_body(x_hbm, pos_hbm, par_hbm,
                    sel_hbm, tgt_hbm, ps_hbm, cs_hbm, cb_hbm,
                    xv0, xv1, pv, parv, wsp, psv, csv, cbv, selv, tgtv,
                    sem, sem_out):
    wid = lax.axis_index("s") * NC + lax.axis_index("c")
    b = wid // (ST // BPW)
    st0 = (wid % (ST // BPW)) * BPW

    cps = [
        pltpu.async_copy(par_hbm, parv, sem),
        pltpu.async_copy(x_hbm.at[pl.ds((b * 2 * ST + st0) * (8 * BLK),
                                        BPW * 8 * BLK)], xv0, sem),
        pltpu.async_copy(x_hbm.at[pl.ds(((b * 2 + 1) * ST + st0) * (8 * BLK),
                                        BPW * 8 * BLK)], xv1, sem),
    ]
    for i in range(BPW):
        cps.append(pltpu.async_copy(
            pos_hbm.at[pl.ds(((st0 + i) * B + b) * BLK, BLK)],
            pv.at[pl.ds(i * BLK, BLK)], sem))
    for cp in cps:
        cp.wait()

    def prow(i):
        return parv[pl.ds(i * L, L)]

    thr = prow(0)
    pw = prow(1)
    cw = prow(2)
    pearly = prow(3)
    plate = prow(4)

    wsc = []
    for t in range(T):
        wt = prow(5 + t)
        wsc.append([wt[c] for c in range(D)])
    for t in range(WSPLIT):
        for c in range(D):
            wsp[pl.ds((t * D + c) * L, L)] = jnp.full((L,), wsc[t][c])

    esp, lsp = [], []
    for t in range(T):
        wt = prow(5 + T + t)
        esp.append(jnp.full((L,), jnp.sum(wt * pearly)))
        lsp.append(jnp.full((L,), jnp.sum(wt * plate)))

    def block(i, carry):
        def group(q, gcarry):
            xb = i * (8 * BLK) + q * L
            pvec = pv[pl.ds(i * BLK + q * L, L)]
            mask = pvec.astype(jnp.float32) < thr

            xT0 = xv0[pl.ds(xb, L)]
            xT1 = xv0[pl.ds(xb + BLK, L)]
            xR = []
            for c in range(D):
                src = xv0 if c < 8 else xv1
                xR.append(_bf16_round(src[pl.ds(xb + (c % 8) * BLK, L)]))

            ob = i * (T * BLK) + q * L
            best = None
            bidx = None
            for t in range(T):
                if t < WSPLIT:
                    acc = xR[0] * wsp[pl.ds((t * D) * L, L)]
                    for c in range(1, D):
                        acc = acc + xR[c] * wsp[pl.ds((t * D + c) * L, L)]
                else:
                    acc = xR[0] * wsc[t][0]
                    for c in range(1, D):
                        acc = acc + xR[c] * wsc[t][c]
                post = jnp.where(mask, esp[t], lsp[t])
                comb = pw * post + cw * acc
                psv[pl.ds(ob + t * BLK, L)] = post
                csv[pl.ds(ob + t * BLK, L)] = acc
                cbv[pl.ds(ob + t * BLK, L)] = comb
                if t == 0:
                    best, bidx = comb, _full(0)
                else:
                    gt = comb > best
                    best = jnp.where(gt, comb, best)
                    bidx = jnp.where(gt, _full(t), bidx)

            pos_class = jnp.where(mask, _full(0), _full(1))
            f0 = (xT0 > 0).astype(jnp.int32)
            f1 = (xT1 > 0).astype(jnp.int32)
            selv[pl.ds(i * BLK + q * L, L)] = bidx
            tgtv[pl.ds(i * BLK + q * L, L)] = pos_class * 4 + f0 * 2 + f1
            return gcarry

        lax.fori_loop(0, NG, group, 0)

        off = ((st0 + i) * B + b) * BLK
        sc_off = (wid * BPW + i) * (T * BLK)
        for src, dst in ((psv, ps_hbm), (csv, cs_hbm), (cbv, cb_hbm)):
            pltpu.async_copy(src.at[pl.ds(i * (T * BLK), T * BLK)],
                             dst.at[pl.ds(sc_off, T * BLK)], sem_out)
        pltpu.async_copy(selv.at[pl.ds(i * BLK, BLK)],
                         sel_hbm.at[pl.ds(off, BLK)], sem_out)
        pltpu.async_copy(tgtv.at[pl.ds(i * BLK, BLK)],
                         tgt_hbm.at[pl.ds(off, BLK)], sem_out)
        return carry

    lax.fori_loop(0, BPW, block, 0)

    for i in range(BPW):
        off = ((st0 + i) * B + b) * BLK
        sc_off = (wid * BPW + i) * (T * BLK)
        for src, dst in ((psv, ps_hbm), (csv, cs_hbm), (cbv, cb_hbm)):
            pltpu.make_async_copy(src.at[pl.ds(i * (T * BLK), T * BLK)],
                                  dst.at[pl.ds(sc_off, T * BLK)],
                                  sem_out).wait()
        pltpu.make_async_copy(selv.at[pl.ds(i * BLK, BLK)],
                              sel_hbm.at[pl.ds(off, BLK)], sem_out).wait()
        pltpu.make_async_copy(tgtv.at[pl.ds(i * BLK, BLK)],
                              tgt_hbm.at[pl.ds(off, BLK)], sem_out).wait()


_OUT_TYPE = (
    jax.ShapeDtypeStruct((N_TOK,), jnp.int32),
    jax.ShapeDtypeStruct((N_TOK,), jnp.int32),
    jax.ShapeDtypeStruct((N_TOK * T,), jnp.float32),
    jax.ShapeDtypeStruct((N_TOK * T,), jnp.float32),
    jax.ShapeDtypeStruct((N_TOK * T,), jnp.float32),
)

_SCRATCH = (
    pltpu.VMEM((BPW * 8 * BLK,), jnp.float32),
    pltpu.VMEM((BPW * 8 * BLK,), jnp.float32),
    pltpu.VMEM((BPW * BLK,), jnp.int32),
    pltpu.VMEM((NPAR * L,), jnp.float32),
    pltpu.VMEM((WSPLIT * D * L,), jnp.float32),
    pltpu.VMEM((BPW * T * BLK,), jnp.float32),
    pltpu.VMEM((BPW * T * BLK,), jnp.float32),
    pltpu.VMEM((BPW * T * BLK,), jnp.float32),
    pltpu.VMEM((BPW * BLK,), jnp.int32),
    pltpu.VMEM((BPW * BLK,), jnp.int32),
    pltpu.SemaphoreType.DMA,
    pltpu.SemaphoreType.DMA,
)


@functools.lru_cache(maxsize=None)
def _sc_router():
    return pl.kernel(
        _sc_router_body,
        out_type=_OUT_TYPE,
        mesh=plsc.VectorSubcoreMesh(core_axis_name="c", subcore_axis_name="s",
                                    num_cores=NC, num_subcores=NS),
        scratch_types=_SCRATCH,
        compiler_params=pltpu.CompilerParams(
            needs_layout_passes=False,
            disable_bounds_checks=True,
            disable_semaphore_checks=True,
            skip_device_barrier=True,
        ),
    )


def _b16(v):
    y = lax.bitcast_convert_type(v, jnp.int32)
    odd = lax.shift_right_logical(y, 16) & 1
    r = (y + 32767 + odd) & (-65536)
    return lax.bitcast_convert_type(r, jnp.float32)


def kernel(x, positions, seq_len, position_sigs, content_sigs,
           position_logit, content_logit, pos_early, pos_late):
    xf = (x.astype(jnp.float32)
          .transpose(0, 2, 1)
          .reshape(B, 2, 8, ST, BLK)
          .transpose(0, 1, 3, 2, 4)
          .reshape(N_TOK * D))
    pf = (positions.astype(jnp.int32)
          .reshape(B, ST, BLK)
          .transpose(1, 0, 2)
          .reshape(N_TOK))
    half = jnp.asarray(seq_len, jnp.float32) / 2.0
    sp = jax.nn.sigmoid(jnp.asarray(position_logit, jnp.float32))
    sc = jax.nn.sigmoid(jnp.asarray(content_logit, jnp.float32))
    raw = jnp.concatenate([
        jnp.full((L,), half, jnp.float32),
        jnp.full((L,), sp / (sp + sc), jnp.float32),
        jnp.full((L,), sc / (sp + sc), jnp.float32),
        pos_early.astype(jnp.float32),
        pos_late.astype(jnp.float32),
        content_sigs.astype(jnp.float32).reshape(-1),
        position_sigs.astype(jnp.float32).reshape(-1),
    ])
    row = lax.iota(jnp.int32, NPAR * L) // L
    v = jnp.where(row >= 5, jnp.tanh(raw), raw)
    params = jnp.where(row >= 3, _b16(v), v)

    sel, tgt, ps, cs, cb = _sc_router()(xf, pf, params)

    def untile_tok(v):
        return v.reshape(ST, B, BLK).transpose(1, 0, 2).reshape(B, S)

    def untile_scores(v):
        return (v.reshape(B, ST, T, BLK)
                .transpose(0, 1, 3, 2)
                .reshape(B, S, T))

    return (untile_tok(sel), untile_tok(tgt),
            untile_scores(ps), untile_scores(cs), untile_scores(cb))


# speedup vs baseline: 1.2522x; 1.1228x over previous
import functools

import jax
import jax.numpy as jnp
from jax import lax
from jax.experimental import pallas as pl
from jax.experimental.pallas import tpu as pltpu
from jax.experimental.pallas import tpu_sc as plsc

L = 16
NC = 2
NS = 16
NW = NC * NS
T = 8
D = 16
B = 4
S = 8192
N_TOK = B * S
BLK = 128
NBLK = N_TOK // BLK
BPW = NBLK // NW
NG = BLK // L
ST = S // BLK
NPAR = 5 + 2 * T
WSPLIT = 8


def _full(v, dtype=jnp.int32):
    return jnp.full((L,), v, dtype)


def _bf16_round(v):
    t = v * 65537.0
    return t - (t - v)


def _sc_router_body(x_hbm, pos_hbm, par_hbm,
                    sel_hbm, tgt_hbm, ps_hbm, cs_hbm, cb_hbm,
                    xv0, xv1, pv, parv, wsp, psv, csv, cbv, selv, tgtv,
                    sem, sem_out):
    wid = lax.axis_index("s") * NC + lax.axis_index("c")
    b = wid // (ST // BPW)
    st0 = (wid % (ST // BPW)) * BPW

    cps = [
        pltpu.async_copy(par_hbm, parv, sem),
        pltpu.async_copy(x_hbm.at[pl.ds((b * 2 * ST + st0) * (8 * BLK),
                                        BPW * 8 * BLK)], xv0, sem),
        pltpu.async_copy(x_hbm.at[pl.ds(((b * 2 + 1) * ST + st0) * (8 * BLK),
                                        BPW * 8 * BLK)], xv1, sem),
    ]
    for i in range(BPW):
        cps.append(pltpu.async_copy(
            pos_hbm.at[pl.ds(((st0 + i) * B + b) * BLK, BLK)],
            pv.at[pl.ds(i * BLK, BLK)], sem))
    for cp in cps:
        cp.wait()

    def prow(i):
        return parv[pl.ds(i * L, L)]

    thr = prow(0)
    pw = prow(1)
    cw = prow(2)
    pearly = prow(3)
    plate = prow(4)

    wsc = []
    for t in range(T):
        wt = prow(5 + t)
        wsc.append([wt[c] for c in range(D)])
    for t in range(WSPLIT):
        for c in range(D):
            wsp[pl.ds((t * D + c) * L, L)] = jnp.full((L,), wsc[t][c])

    esp, lsp = [], []
    for t in range(T):
        wt = prow(5 + T + t)
        esp.append(jnp.full((L,), jnp.sum(wt * pearly)))
        lsp.append(jnp.full((L,), jnp.sum(wt * plate)))

    def block(i, carry):
        def pair(j, gcarry):
            xbs = [i * (8 * BLK) + (2 * j + h) * L for h in range(2)]
            masks = []
            xRs = []
            for h in range(2):
                pvec = pv[pl.ds(i * BLK + (2 * j + h) * L, L)]
                masks.append(pvec.astype(jnp.float32) < thr)
                xRs.append([_bf16_round(
                    (xv0 if c < 8 else xv1)[pl.ds(xbs[h] + (c % 8) * BLK, L)])
                    for c in range(D)])

            obs = [i * (T * BLK) + (2 * j + h) * L for h in range(2)]
            best = [None, None]
            bidx = [None, None]
            for t in range(T):
                w0 = wsp[pl.ds((t * D) * L, L)]
                accs = [xRs[0][0] * w0, xRs[1][0] * w0]
                for c in range(1, D):
                    wv = wsp[pl.ds((t * D + c) * L, L)]
                    accs[0] = accs[0] + xRs[0][c] * wv
                    accs[1] = accs[1] + xRs[1][c] * wv
                for h in range(2):
                    post = jnp.where(masks[h], esp[t], lsp[t])
                    comb = pw * post + cw * accs[h]
                    psv[pl.ds(obs[h] + t * BLK, L)] = post
                    csv[pl.ds(obs[h] + t * BLK, L)] = accs[h]
                    cbv[pl.ds(obs[h] + t * BLK, L)] = comb
                    if t == 0:
                        best[h], bidx[h] = comb, _full(0)
                    else:
                        gt = comb > best[h]
                        best[h] = jnp.where(gt, comb, best[h])
                        bidx[h] = jnp.where(gt, _full(t), bidx[h])

            for h in range(2):
                pos_class = jnp.where(masks[h], _full(0), _full(1))
                f0 = (xRs[h][0] > 0).astype(jnp.int32)
                f1 = (xRs[h][1] > 0).astype(jnp.int32)
                selv[pl.ds(i * BLK + (2 * j + h) * L, L)] = bidx[h]
                tgtv[pl.ds(i * BLK + (2 * j + h) * L, L)] = \
                    pos_class * 4 + f0 * 2 + f1
            return gcarry

        lax.fori_loop(0, NG // 2, pair, 0)

        off = ((st0 + i) * B + b) * BLK
        sc_off = (wid * BPW + i) * (T * BLK)
        for src, dst in ((psv, ps_hbm), (csv, cs_hbm), (cbv, cb_hbm)):
            pltpu.async_copy(src.at[pl.ds(i * (T * BLK), T * BLK)],
                             dst.at[pl.ds(sc_off, T * BLK)], sem_out)
        pltpu.async_copy(selv.at[pl.ds(i * BLK, BLK)],
                         sel_hbm.at[pl.ds(off, BLK)], sem_out)
        pltpu.async_copy(tgtv.at[pl.ds(i * BLK, BLK)],
                         tgt_hbm.at[pl.ds(off, BLK)], sem_out)
        return carry

    lax.fori_loop(0, BPW, block, 0)

    for i in range(BPW):
        off = ((st0 + i) * B + b) * BLK
        sc_off = (wid * BPW + i) * (T * BLK)
        for src, dst in ((psv, ps_hbm), (csv, cs_hbm), (cbv, cb_hbm)):
            pltpu.make_async_copy(src.at[pl.ds(i * (T * BLK), T * BLK)],
                                  dst.at[pl.ds(sc_off, T * BLK)],
                                  sem_out).wait()
        pltpu.make_async_copy(selv.at[pl.ds(i * BLK, BLK)],
                              sel_hbm.at[pl.ds(off, BLK)], sem_out).wait()
        pltpu.make_async_copy(tgtv.at[pl.ds(i * BLK, BLK)],
                              tgt_hbm.at[pl.ds(off, BLK)], sem_out).wait()


_OUT_TYPE = (
    jax.ShapeDtypeStruct((N_TOK,), jnp.int32),
    jax.ShapeDtypeStruct((N_TOK,), jnp.int32),
    jax.ShapeDtypeStruct((N_TOK * T,), jnp.float32),
    jax.ShapeDtypeStruct((N_TOK * T,), jnp.float32),
    jax.ShapeDtypeStruct((N_TOK * T,), jnp.float32),
)

_SCRATCH = (
    pltpu.VMEM((BPW * 8 * BLK,), jnp.float32),
    pltpu.VMEM((BPW * 8 * BLK,), jnp.float32),
    pltpu.VMEM((BPW * BLK,), jnp.int32),
    pltpu.VMEM((NPAR * L,), jnp.float32),
    pltpu.VMEM((WSPLIT * D * L,), jnp.float32),
    pltpu.VMEM((BPW * T * BLK,), jnp.float32),
    pltpu.VMEM((BPW * T * BLK,), jnp.float32),
    pltpu.VMEM((BPW * T * BLK,), jnp.float32),
    pltpu.VMEM((BPW * BLK,), jnp.int32),
    pltpu.VMEM((BPW * BLK,), jnp.int32),
    pltpu.SemaphoreType.DMA,
    pltpu.SemaphoreType.DMA,
)


@functools.lru_cache(maxsize=None)
def _sc_router():
    return pl.kernel(
        _sc_router_body,
        out_type=_OUT_TYPE,
        mesh=plsc.VectorSubcoreMesh(core_axis_name="c", subcore_axis_name="s",
                                    num_cores=NC, num_subcores=NS),
        scratch_types=_SCRATCH,
        compiler_params=pltpu.CompilerParams(
            needs_layout_passes=False,
            disable_bounds_checks=True,
            disable_semaphore_checks=True,
            skip_device_barrier=True,
        ),
    )


def _b16(v):
    y = lax.bitcast_convert_type(v, jnp.int32)
    odd = lax.shift_right_logical(y, 16) & 1
    r = (y + 32767 + odd) & (-65536)
    return lax.bitcast_convert_type(r, jnp.float32)


def kernel(x, positions, seq_len, position_sigs, content_sigs,
           position_logit, content_logit, pos_early, pos_late):
    xf = (x.astype(jnp.float32)
          .transpose(0, 2, 1)
          .reshape(B, 2, 8, ST, BLK)
          .transpose(0, 1, 3, 2, 4)
          .reshape(N_TOK * D))
    pf = (positions.astype(jnp.int32)
          .reshape(B, ST, BLK)
          .transpose(1, 0, 2)
          .reshape(N_TOK))
    half = jnp.asarray(seq_len, jnp.float32) / 2.0
    sp = jax.nn.sigmoid(jnp.asarray(position_logit, jnp.float32))
    sc = jax.nn.sigmoid(jnp.asarray(content_logit, jnp.float32))
    raw = jnp.concatenate([
        jnp.full((L,), half, jnp.float32),
        jnp.full((L,), sp / (sp + sc), jnp.float32),
        jnp.full((L,), sc / (sp + sc), jnp.float32),
        pos_early.astype(jnp.float32),
        pos_late.astype(jnp.float32),
        content_sigs.astype(jnp.float32).reshape(-1),
        position_sigs.astype(jnp.float32).reshape(-1),
    ])
    row = lax.iota(jnp.int32, NPAR * L) // L
    v = jnp.where(row >= 5, jnp.tanh(raw), raw)
    params = jnp.where(row >= 3, _b16(v), v)

    sel, tgt, ps, cs, cb = _sc_router()(xf, pf, params)

    def untile_tok(v):
        return v.reshape(ST, B, BLK).transpose(1, 0, 2).reshape(B, S)

    def untile_scores(v):
        return (v.reshape(B, ST, T, BLK)
                .transpose(0, 1, 3, 2)
                .reshape(B, S, T))

    return (untile_tok(sel), untile_tok(tgt),
            untile_scores(ps), untile_scores(cs), untile_scores(cb))


# speedup vs baseline: 1.2929x; 1.0325x over previous
import functools

import jax
import jax.numpy as jnp
from jax import lax
from jax.experimental import pallas as pl
from jax.experimental.pallas import tpu as pltpu
from jax.experimental.pallas import tpu_sc as plsc

L = 16
NC = 2
NS = 16
NW = NC * NS
T = 8
D = 16
B = 4
S = 8192
N_TOK = B * S
BLK = 128
NBLK = N_TOK // BLK
BPW = NBLK // NW
NG = BLK // L
ST = S // BLK
NPAR = 5 + 2 * T
WSPLIT = 8
GW = 4


def _full(v, dtype=jnp.int32):
    return jnp.full((L,), v, dtype)


def _bf16_round(v):
    t = v * 65537.0
    return t - (t - v)


def _sc_router_body(x_hbm, pos_hbm, par_hbm,
                    sel_hbm, tgt_hbm, ps_hbm, cs_hbm, cb_hbm,
                    xv0, xv1, pv, parv, wsp, psv, csv, cbv, selv, tgtv,
                    sem, sem_out):
    wid = lax.axis_index("s") * NC + lax.axis_index("c")
    b = wid // (ST // BPW)
    st0 = (wid % (ST // BPW)) * BPW

    cps = [
        pltpu.async_copy(par_hbm, parv, sem),
        pltpu.async_copy(x_hbm.at[pl.ds((b * 2 * ST + st0) * (8 * BLK),
                                        BPW * 8 * BLK)], xv0, sem),
        pltpu.async_copy(x_hbm.at[pl.ds(((b * 2 + 1) * ST + st0) * (8 * BLK),
                                        BPW * 8 * BLK)], xv1, sem),
    ]
    for i in range(BPW):
        cps.append(pltpu.async_copy(
            pos_hbm.at[pl.ds(((st0 + i) * B + b) * BLK, BLK)],
            pv.at[pl.ds(i * BLK, BLK)], sem))
    for cp in cps:
        cp.wait()

    def prow(i):
        return parv[pl.ds(i * L, L)]

    thr = prow(0)
    pw = prow(1)
    cw = prow(2)
    pearly = prow(3)
    plate = prow(4)

    wsc = []
    for t in range(T):
        wt = prow(5 + t)
        wsc.append([wt[c] for c in range(D)])
    for t in range(WSPLIT):
        for c in range(D):
            wsp[pl.ds((t * D + c) * L, L)] = jnp.full((L,), wsc[t][c])

    esp, lsp = [], []
    for t in range(T):
        wt = prow(5 + T + t)
        esp.append(jnp.full((L,), jnp.sum(wt * pearly)))
        lsp.append(jnp.full((L,), jnp.sum(wt * plate)))

    def block(i, carry):
        def pair(j, gcarry):
            xbs = [i * (8 * BLK) + (GW * j + h) * L for h in range(GW)]
            masks = []
            xRs = []
            for h in range(GW):
                pvec = pv[pl.ds(i * BLK + (GW * j + h) * L, L)]
                masks.append(pvec.astype(jnp.float32) < thr)
                xRs.append([_bf16_round(
                    (xv0 if c < 8 else xv1)[pl.ds(xbs[h] + (c % 8) * BLK, L)])
                    for c in range(D)])

            obs = [i * (T * BLK) + (GW * j + h) * L for h in range(GW)]
            best = [None] * GW
            bidx = [None] * GW
            for t in range(T):
                w0 = wsp[pl.ds((t * D) * L, L)]
                accs = [xRs[h][0] * w0 for h in range(GW)]
                for c in range(1, D):
                    wv = wsp[pl.ds((t * D + c) * L, L)]
                    for h in range(GW):
                        accs[h] = accs[h] + xRs[h][c] * wv
                for h in range(GW):
                    post = jnp.where(masks[h], esp[t], lsp[t])
                    comb = pw * post + cw * accs[h]
                    psv[pl.ds(obs[h] + t * BLK, L)] = post
                    csv[pl.ds(obs[h] + t * BLK, L)] = accs[h]
                    cbv[pl.ds(obs[h] + t * BLK, L)] = comb
                    if t == 0:
                        best[h], bidx[h] = comb, _full(0)
                    else:
                        gt = comb > best[h]
                        best[h] = jnp.where(gt, comb, best[h])
                        bidx[h] = jnp.where(gt, _full(t), bidx[h])

            for h in range(GW):
                pos_class = jnp.where(masks[h], _full(0), _full(1))
                f0 = (xRs[h][0] > 0).astype(jnp.int32)
                f1 = (xRs[h][1] > 0).astype(jnp.int32)
                selv[pl.ds(i * BLK + (GW * j + h) * L, L)] = bidx[h]
                tgtv[pl.ds(i * BLK + (GW * j + h) * L, L)] = \
                    pos_class * 4 + f0 * 2 + f1
            return gcarry

        lax.fori_loop(0, NG // GW, pair, 0)

        off = ((st0 + i) * B + b) * BLK
        sc_off = (wid * BPW + i) * (T * BLK)
        for src, dst in ((psv, ps_hbm), (csv, cs_hbm), (cbv, cb_hbm)):
            pltpu.async_copy(src.at[pl.ds(i * (T * BLK), T * BLK)],
                             dst.at[pl.ds(sc_off, T * BLK)], sem_out)
        pltpu.async_copy(selv.at[pl.ds(i * BLK, BLK)],
                         sel_hbm.at[pl.ds(off, BLK)], sem_out)
        pltpu.async_copy(tgtv.at[pl.ds(i * BLK, BLK)],
                         tgt_hbm.at[pl.ds(off, BLK)], sem_out)
        return carry

    lax.fori_loop(0, BPW, block, 0)

    for i in range(BPW):
        off = ((st0 + i) * B + b) * BLK
        sc_off = (wid * BPW + i) * (T * BLK)
        for src, dst in ((psv, ps_hbm), (csv, cs_hbm), (cbv, cb_hbm)):
            pltpu.make_async_copy(src.at[pl.ds(i * (T * BLK), T * BLK)],
                                  dst.at[pl.ds(sc_off, T * BLK)],
                                  sem_out).wait()
        pltpu.make_async_copy(selv.at[pl.ds(i * BLK, BLK)],
                              sel_hbm.at[pl.ds(off, BLK)], sem_out).wait()
        pltpu.make_async_copy(tgtv.at[pl.ds(i * BLK, BLK)],
                              tgt_hbm.at[pl.ds(off, BLK)], sem_out).wait()


_OUT_TYPE = (
    jax.ShapeDtypeStruct((N_TOK,), jnp.int32),
    jax.ShapeDtypeStruct((N_TOK,), jnp.int32),
    jax.ShapeDtypeStruct((N_TOK * T,), jnp.float32),
    jax.ShapeDtypeStruct((N_TOK * T,), jnp.float32),
    jax.ShapeDtypeStruct((N_TOK * T,), jnp.float32),
)

_SCRATCH = (
    pltpu.VMEM((BPW * 8 * BLK,), jnp.float32),
    pltpu.VMEM((BPW * 8 * BLK,), jnp.float32),
    pltpu.VMEM((BPW * BLK,), jnp.int32),
    pltpu.VMEM((NPAR * L,), jnp.float32),
    pltpu.VMEM((WSPLIT * D * L,), jnp.float32),
    pltpu.VMEM((BPW * T * BLK,), jnp.float32),
    pltpu.VMEM((BPW * T * BLK,), jnp.float32),
    pltpu.VMEM((BPW * T * BLK,), jnp.float32),
    pltpu.VMEM((BPW * BLK,), jnp.int32),
    pltpu.VMEM((BPW * BLK,), jnp.int32),
    pltpu.SemaphoreType.DMA,
    pltpu.SemaphoreType.DMA,
)


@functools.lru_cache(maxsize=None)
def _sc_router():
    return pl.kernel(
        _sc_router_body,
        out_type=_OUT_TYPE,
        mesh=plsc.VectorSubcoreMesh(core_axis_name="c", subcore_axis_name="s",
                                    num_cores=NC, num_subcores=NS),
        scratch_types=_SCRATCH,
        compiler_params=pltpu.CompilerParams(
            needs_layout_passes=False,
            disable_bounds_checks=True,
            disable_semaphore_checks=True,
            skip_device_barrier=True,
        ),
    )


def _b16(v):
    y = lax.bitcast_convert_type(v, jnp.int32)
    odd = lax.shift_right_logical(y, 16) & 1
    r = (y + 32767 + odd) & (-65536)
    return lax.bitcast_convert_type(r, jnp.float32)


def kernel(x, positions, seq_len, position_sigs, content_sigs,
           position_logit, content_logit, pos_early, pos_late):
    xf = (x.astype(jnp.float32)
          .transpose(0, 2, 1)
          .reshape(B, 2, 8, ST, BLK)
          .transpose(0, 1, 3, 2, 4)
          .reshape(N_TOK * D))
    pf = (positions.astype(jnp.int32)
          .reshape(B, ST, BLK)
          .transpose(1, 0, 2)
          .reshape(N_TOK))
    half = jnp.asarray(seq_len, jnp.float32) / 2.0
    sp = jax.nn.sigmoid(jnp.asarray(position_logit, jnp.float32))
    sc = jax.nn.sigmoid(jnp.asarray(content_logit, jnp.float32))
    raw = jnp.concatenate([
        jnp.full((L,), half, jnp.float32),
        jnp.full((L,), sp / (sp + sc), jnp.float32),
        jnp.full((L,), sc / (sp + sc), jnp.float32),
        pos_early.astype(jnp.float32),
        pos_late.astype(jnp.float32),
        content_sigs.astype(jnp.float32).reshape(-1),
        position_sigs.astype(jnp.float32).reshape(-1),
    ])
    row = lax.iota(jnp.int32, NPAR * L) // L
    v = jnp.where(row >= 5, jnp.tanh(raw), raw)
    params = jnp.where(row >= 3, _b16(v), v)

    sel, tgt, ps, cs, cb = _sc_router()(xf, pf, params)

    def untile_tok(v):
        return v.reshape(ST, B, BLK).transpose(1, 0, 2).reshape(B, S)

    def untile_scores(v):
        return (v.reshape(B, ST, T, BLK)
                .transpose(0, 1, 3, 2)
                .reshape(B, S, T))

    return (untile_tok(sel), untile_tok(tgt),
            untile_scores(ps), untile_scores(cs), untile_scores(cb))
